# Initial kernel scaffold; baseline (speedup 1.0000x reference)
import jax
import jax.numpy as jnp
from jax.experimental import pallas as pl


def kernel(x, neigh_indices):
    raise NotImplementedError("write your pallas kernel here")


# speedup vs baseline: 4.6029x; 4.6029x over previous
import functools

import jax
import jax.numpy as jnp
from jax import lax
from jax.experimental import pallas as pl
from jax.experimental.pallas import tpu as pltpu
from jax.experimental.pallas import tpu_sc as plsc

N = 25000
D = 128
K = 7
NC, NS = 2, 16
NW = NC * NS
ROWS_PER_CHUNK = 16
IDX_PER_CHUNK = ROWS_PER_CHUNK * K
CHUNKS_PER_WORKER = 49
N_PAD = NW * CHUNKS_PER_WORKER * ROWS_PER_CHUNK


def _sc_body(x_hbm, idx_hbm, out_hbm, idx_v, gbuf, obuf, sem):
    wid = lax.axis_index("s") * NC + lax.axis_index("c")
    base_chunk = wid * CHUNKS_PER_WORKER
    idx_per_worker = CHUNKS_PER_WORKER * IDX_PER_CHUNK
    pltpu.sync_copy(idx_hbm.at[pl.ds(wid * idx_per_worker, idx_per_worker)], idx_v)

    def chunk_body(c, carry):
        pltpu.async_copy(
            x_hbm.at[idx_v.at[pl.ds(c * IDX_PER_CHUNK, IDX_PER_CHUNK)]], gbuf, sem
        ).wait()

        def row_body(i, carry2):
            for g in range(D // 16):
                s = pl.ds(g * 16, 16)
                m = gbuf[i * K, s]
                for k in range(1, K):
                    m = jnp.maximum(m, gbuf[i * K + k, s])
                obuf[i, s] = m
            return carry2

        lax.fori_loop(0, ROWS_PER_CHUNK, row_body, 0)
        pltpu.sync_copy(
            obuf,
            out_hbm.at[pl.ds((base_chunk + c) * ROWS_PER_CHUNK, ROWS_PER_CHUNK), :],
        )
        return carry

    lax.fori_loop(0, CHUNKS_PER_WORKER, chunk_body, 0)


@jax.jit
def _hex_pool(x, idx2d):
    mesh = plsc.VectorSubcoreMesh(
        core_axis_name="c", subcore_axis_name="s", num_cores=NC, num_subcores=NS
    )
    return pl.kernel(
        _sc_body,
        out_type=jax.ShapeDtypeStruct((N_PAD, D), jnp.float32),
        mesh=mesh,
        scratch_types=[
            pltpu.VMEM((CHUNKS_PER_WORKER * IDX_PER_CHUNK,), jnp.int32),
            pltpu.VMEM((IDX_PER_CHUNK, D), jnp.float32),
            pltpu.VMEM((ROWS_PER_CHUNK, D), jnp.float32),
            pltpu.SemaphoreType.DMA,
        ],
    )(x, idx2d)


def kernel(x, neigh_indices):
    ni = neigh_indices.astype(jnp.int32).reshape(-1)
    pad = N_PAD * K - ni.shape[0]
    ni = jnp.concatenate([ni, jnp.zeros((pad,), jnp.int32)])
    out = _hex_pool(x, ni)
    return out[:N]


# speedup vs baseline: 6.6372x; 1.4420x over previous
import functools

import jax
import jax.numpy as jnp
from jax import lax
from jax.experimental import pallas as pl
from jax.experimental.pallas import tpu as pltpu
from jax.experimental.pallas import tpu_sc as plsc

N = 25000
D = 128
K = 7
NC, NS = 2, 16
NW = NC * NS
ROWS_PER_CHUNK = 16
IDX_PER_CHUNK = ROWS_PER_CHUNK * K
CHUNKS_PER_WORKER = 49
N_PAD = NW * CHUNKS_PER_WORKER * ROWS_PER_CHUNK


def _sc_body(x_hbm, idx_hbm, out_hbm, idx_v, gbuf, obuf, gsem, osem):
    wid = lax.axis_index("s") * NC + lax.axis_index("c")
    base_chunk = wid * CHUNKS_PER_WORKER
    idx_per_worker = CHUNKS_PER_WORKER * IDX_PER_CHUNK
    pltpu.sync_copy(idx_hbm.at[pl.ds(wid * idx_per_worker, idx_per_worker)], idx_v)

    def start_gather(c, slot):
        pltpu.async_copy(
            x_hbm.at[idx_v.at[pl.ds(c * IDX_PER_CHUNK, IDX_PER_CHUNK)]],
            gbuf.at[slot],
            gsem.at[slot],
        )

    def out_slice(c):
        return out_hbm.at[pl.ds((base_chunk + c) * ROWS_PER_CHUNK, ROWS_PER_CHUNK), :]

    start_gather(0, 0)

    def chunk_body(c, carry):
        slot = lax.rem(c, 2)
        nslot = 1 - slot

        @pl.when(c + 1 < CHUNKS_PER_WORKER)
        def _():
            start_gather(c + 1, nslot)

        pltpu.make_async_copy(
            x_hbm.at[pl.ds(0, IDX_PER_CHUNK), :], gbuf.at[slot], gsem.at[slot]
        ).wait()

        @pl.when(c >= 2)
        def _():
            pltpu.make_async_copy(obuf.at[slot], out_slice(c), osem.at[slot]).wait()

        def row_body(i, carry2):
            for g in range(D // 16):
                s = pl.ds(g * 16, 16)
                m = gbuf[slot, i * K, s]
                for k in range(1, K):
                    m = jnp.maximum(m, gbuf[slot, i * K + k, s])
                obuf[slot, i, s] = m
            return carry2

        lax.fori_loop(0, ROWS_PER_CHUNK, row_body, 0)
        pltpu.async_copy(obuf.at[slot], out_slice(c), osem.at[slot])
        return carry

    lax.fori_loop(0, CHUNKS_PER_WORKER, chunk_body, 0)
    for slot in range(2):
        pltpu.make_async_copy(obuf.at[slot], out_slice(0), osem.at[slot]).wait()


@jax.jit
def _hex_pool(x, idx2d):
    mesh = plsc.VectorSubcoreMesh(
        core_axis_name="c", subcore_axis_name="s", num_cores=NC, num_subcores=NS
    )
    return pl.kernel(
        _sc_body,
        out_type=jax.ShapeDtypeStruct((N_PAD, D), jnp.float32),
        mesh=mesh,
        scratch_types=[
            pltpu.VMEM((CHUNKS_PER_WORKER * IDX_PER_CHUNK,), jnp.int32),
            pltpu.VMEM((2, IDX_PER_CHUNK, D), jnp.float32),
            pltpu.VMEM((2, ROWS_PER_CHUNK, D), jnp.float32),
            pltpu.SemaphoreType.DMA((2,)),
            pltpu.SemaphoreType.DMA((2,)),
        ],
    )(x, idx2d)


def kernel(x, neigh_indices):
    ni = neigh_indices.astype(jnp.int32).reshape(-1)
    pad = N_PAD * K - ni.shape[0]
    ni = jnp.concatenate([ni, jnp.zeros((pad,), jnp.int32)])
    out = _hex_pool(x, ni)
    return out[:N]


# speedup vs baseline: 7.1006x; 1.0698x over previous
import functools

import jax
import jax.numpy as jnp
from jax import lax
from jax.experimental import pallas as pl
from jax.experimental.pallas import tpu as pltpu
from jax.experimental.pallas import tpu_sc as plsc

N = 25000
D = 128
K = 7
NC, NS = 2, 16
NW = NC * NS
ROWS_PER_CHUNK = 16
IDX_PER_CHUNK = ROWS_PER_CHUNK * K
CHUNKS_PER_WORKER = 49
N_PAD = NW * CHUNKS_PER_WORKER * ROWS_PER_CHUNK
NBUF = 4


def _sc_body(x_hbm, idx_hbm, out_hbm, idx_v, gbuf, obuf, gsem, osem):
    wid = lax.axis_index("s") * NC + lax.axis_index("c")
    base_chunk = wid * CHUNKS_PER_WORKER
    idx_per_worker = CHUNKS_PER_WORKER * IDX_PER_CHUNK
    pltpu.sync_copy(idx_hbm.at[pl.ds(wid * idx_per_worker, idx_per_worker)], idx_v)

    def start_gather(c, slot):
        pltpu.async_copy(
            x_hbm.at[idx_v.at[pl.ds(c * IDX_PER_CHUNK, IDX_PER_CHUNK)]],
            gbuf.at[slot],
            gsem.at[slot],
        )

    def out_slice(c):
        return out_hbm.at[pl.ds((base_chunk + c) * ROWS_PER_CHUNK, ROWS_PER_CHUNK), :]

    for c in range(NBUF - 1):
        start_gather(c, c)

    def chunk_body(c, carry):
        slot = lax.rem(c, NBUF)

        @pl.when(c + NBUF - 1 < CHUNKS_PER_WORKER)
        def _():
            start_gather(c + NBUF - 1, lax.rem(c + NBUF - 1, NBUF))

        pltpu.make_async_copy(
            x_hbm.at[pl.ds(0, IDX_PER_CHUNK), :], gbuf.at[slot], gsem.at[slot]
        ).wait()

        @pl.when(c >= NBUF)
        def _():
            pltpu.make_async_copy(obuf.at[slot], out_slice(c), osem.at[slot]).wait()

        def row_body(i, carry2):
            for g in range(D // 16):
                s = pl.ds(g * 16, 16)
                m = gbuf[slot, i * K, s]
                for k in range(1, K):
                    m = jnp.maximum(m, gbuf[slot, i * K + k, s])
                obuf[slot, i, s] = m
            return carry2

        lax.fori_loop(0, ROWS_PER_CHUNK, row_body, 0)
        pltpu.async_copy(obuf.at[slot], out_slice(c), osem.at[slot])
        return carry

    lax.fori_loop(0, CHUNKS_PER_WORKER, chunk_body, 0)
    for slot in range(NBUF):
        pltpu.make_async_copy(obuf.at[slot], out_slice(0), osem.at[slot]).wait()


@jax.jit
def _hex_pool(x, idx2d):
    mesh = plsc.VectorSubcoreMesh(
        core_axis_name="c", subcore_axis_name="s", num_cores=NC, num_subcores=NS
    )
    return pl.kernel(
        _sc_body,
        out_type=jax.ShapeDtypeStruct((N_PAD, D), jnp.float32),
        mesh=mesh,
        scratch_types=[
            pltpu.VMEM((CHUNKS_PER_WORKER * IDX_PER_CHUNK,), jnp.int32),
            pltpu.VMEM((NBUF, IDX_PER_CHUNK, D), jnp.float32),
            pltpu.VMEM((NBUF, ROWS_PER_CHUNK, D), jnp.float32),
            pltpu.SemaphoreType.DMA((NBUF,)),
            pltpu.SemaphoreType.DMA((NBUF,)),
        ],
    )(x, idx2d)


def kernel(x, neigh_indices):
    ni = neigh_indices.astype(jnp.int32).reshape(-1)
    pad = N_PAD * K - ni.shape[0]
    ni = jnp.concatenate([ni, jnp.zeros((pad,), jnp.int32)])
    out = _hex_pool(x, ni)
    return out[:N]


# speedup vs baseline: 9.7071x; 1.3671x over previous
import functools

import jax
import jax.numpy as jnp
from jax import lax
from jax.experimental import pallas as pl
from jax.experimental.pallas import tpu as pltpu
from jax.experimental.pallas import tpu_sc as plsc

N = 25000
D = 128
K = 7
NC, NS = 2, 16
NW = NC * NS
ROWS_PER_CHUNK = 16
IDX_PER_CHUNK = ROWS_PER_CHUNK * K
CHUNKS_PER_WORKER = 49
NBUF = 4


def _sc_body(x_hbm, idx_hbm, out_hbm, idx_v, gbuf, obuf, gsem, osem):
    wid = lax.axis_index("s") * NC + lax.axis_index("c")
    rows_per_worker = CHUNKS_PER_WORKER * ROWS_PER_CHUNK
    base_row = jnp.minimum(wid * rows_per_worker, N - rows_per_worker)
    idx_per_worker = CHUNKS_PER_WORKER * IDX_PER_CHUNK
    pltpu.sync_copy(idx_hbm.at[pl.ds(base_row * K, idx_per_worker)], idx_v)

    def start_gather(c, slot):
        pltpu.async_copy(
            x_hbm.at[idx_v.at[pl.ds(c * IDX_PER_CHUNK, IDX_PER_CHUNK)]],
            gbuf.at[slot],
            gsem.at[slot],
        )

    def out_slice(c):
        return out_hbm.at[pl.ds(base_row + c * ROWS_PER_CHUNK, ROWS_PER_CHUNK), :]

    for c in range(NBUF - 1):
        start_gather(c, c)

    def chunk_body(c, carry):
        slot = lax.rem(c, NBUF)

        @pl.when(c + NBUF - 1 < CHUNKS_PER_WORKER)
        def _():
            start_gather(c + NBUF - 1, lax.rem(c + NBUF - 1, NBUF))

        pltpu.make_async_copy(
            x_hbm.at[pl.ds(0, IDX_PER_CHUNK), :], gbuf.at[slot], gsem.at[slot]
        ).wait()

        @pl.when(c >= NBUF)
        def _():
            pltpu.make_async_copy(obuf.at[slot], out_slice(c), osem.at[slot]).wait()

        def row_body(i, carry2):
            for g in range(D // 16):
                s = pl.ds(g * 16, 16)
                m = gbuf[slot, i * K, s]
                for k in range(1, K):
                    m = jnp.maximum(m, gbuf[slot, i * K + k, s])
                obuf[slot, i, s] = m
            return carry2

        lax.fori_loop(0, ROWS_PER_CHUNK, row_body, 0)
        pltpu.async_copy(obuf.at[slot], out_slice(c), osem.at[slot])
        return carry

    lax.fori_loop(0, CHUNKS_PER_WORKER, chunk_body, 0)
    for slot in range(NBUF):
        pltpu.make_async_copy(obuf.at[slot], out_slice(0), osem.at[slot]).wait()


@jax.jit
def _hex_pool(x, idx2d):
    mesh = plsc.VectorSubcoreMesh(
        core_axis_name="c", subcore_axis_name="s", num_cores=NC, num_subcores=NS
    )
    return pl.kernel(
        _sc_body,
        out_type=jax.ShapeDtypeStruct((N, D), jnp.float32),
        mesh=mesh,
        scratch_types=[
            pltpu.VMEM((CHUNKS_PER_WORKER * IDX_PER_CHUNK,), jnp.int32),
            pltpu.VMEM((NBUF, IDX_PER_CHUNK, D), jnp.float32),
            pltpu.VMEM((NBUF, ROWS_PER_CHUNK, D), jnp.float32),
            pltpu.SemaphoreType.DMA((NBUF,)),
            pltpu.SemaphoreType.DMA((NBUF,)),
        ],
    )(x, idx2d)


def kernel(x, neigh_indices):
    ni = neigh_indices.astype(jnp.int32).reshape(-1)
    return _hex_pool(x, ni)
